# Initial kernel scaffold; baseline (speedup 1.0000x reference)
import jax
import jax.numpy as jnp
from jax.experimental import pallas as pl


def kernel(features, coords, times, polarities, W, b, gamma, beta):
    raise NotImplementedError("write your pallas kernel here")


# speedup vs baseline: 17.6696x; 17.6696x over previous
import jax
import jax.numpy as jnp
from jax.experimental import pallas as pl
from jax.experimental.pallas import tpu as pltpu

_RATIO = 0.25


def _fps_body(idx0_ref, px_ref, py_ref, pt_ref, out_ref, dist_ref):
    B, N = px_ref.shape
    M = out_ref.shape[1]
    lane_n = jax.lax.broadcasted_iota(jnp.int32, (B, N), 1)
    lane_m = jax.lax.broadcasted_iota(jnp.int32, (B, M), 1)
    idx0 = idx0_ref[...]
    out_ref[...] = jnp.where(lane_m == 0, idx0, 0)
    dist_ref[...] = jnp.full((B, N), jnp.inf, dtype=jnp.float32)

    def body(i, prev_k):
        px = px_ref[...]
        py = py_ref[...]
        pt = pt_ref[...]
        sel = lane_n == prev_k
        lx = jnp.sum(jnp.where(sel, px, 0.0), axis=1, keepdims=True)
        ly = jnp.sum(jnp.where(sel, py, 0.0), axis=1, keepdims=True)
        lt = jnp.sum(jnp.where(sel, pt, 0.0), axis=1, keepdims=True)
        dx = px - lx
        dy = py - ly
        dt = pt - lt
        d = jnp.sqrt(dx * dx + dy * dy + dt * dt + 1e-08)
        dmin = jnp.minimum(dist_ref[...], d)
        dist_ref[...] = dmin
        m = jnp.max(dmin, axis=1, keepdims=True)
        k = jnp.min(jnp.where(dmin == m, lane_n, N), axis=1, keepdims=True)
        k = k.astype(jnp.int32)
        out_ref[...] = jnp.where(lane_m == i, k, out_ref[...])
        return k

    jax.lax.fori_loop(1, M, body, idx0)


def _gather_body(idx_ref, feat_ref, aux_ref, wt_ref, b_ref, g_ref, be_ref,
                 out_ref, auxo_ref, gbuf):
    bi = pl.program_id(0)
    M = out_ref.shape[1]

    def copy_row(r, carry):
        k = idx_ref[bi, r]
        gbuf[pl.ds(r, 1), :] = feat_ref[0, pl.ds(k, 1), :]
        auxo_ref[0, pl.ds(r, 1), :] = aux_ref[0, pl.ds(k, 1), :]
        return carry

    jax.lax.fori_loop(0, M, copy_row, 0, unroll=8)
    feats = gbuf[...]
    proj = jnp.dot(feats, wt_ref[...], preferred_element_type=jnp.float32,
                   precision=jax.lax.Precision.HIGHEST) + b_ref[...]
    mean = jnp.mean(proj, axis=1, keepdims=True)
    var = jnp.mean((proj - mean) ** 2, axis=1, keepdims=True)
    normed = (proj - mean) / jnp.sqrt(var + 1e-05) * g_ref[...] + be_ref[...]
    out_ref[...] = normed[None]


def kernel(features, coords, times, polarities, W, b, gamma, beta):
    B, N, DIN = features.shape
    DOUT = W.shape[0]
    M = max(int(N * _RATIO), 1)

    px = coords[..., 0]
    py = coords[..., 1]
    pt = times
    idx0 = jax.random.randint(jax.random.key(1), (B,), 0, N).astype(jnp.int32)
    idx0 = idx0[:, None]

    indices = pl.pallas_call(
        _fps_body,
        out_shape=jax.ShapeDtypeStruct((B, M), jnp.int32),
        scratch_shapes=[pltpu.VMEM((B, N), jnp.float32)],
    )(idx0, px, py, pt)

    aux = jnp.stack([px, py, pt, polarities], axis=-1)
    wt = W.T
    b2 = b[None, :]
    g2 = gamma[None, :]
    be2 = beta[None, :]

    grid_spec = pltpu.PrefetchScalarGridSpec(
        num_scalar_prefetch=1,
        grid=(B,),
        in_specs=[
            pl.BlockSpec((1, N, DIN), lambda bi, idx: (bi, 0, 0)),
            pl.BlockSpec((1, N, 4), lambda bi, idx: (bi, 0, 0)),
            pl.BlockSpec((DIN, DOUT), lambda bi, idx: (0, 0)),
            pl.BlockSpec((1, DOUT), lambda bi, idx: (0, 0)),
            pl.BlockSpec((1, DOUT), lambda bi, idx: (0, 0)),
            pl.BlockSpec((1, DOUT), lambda bi, idx: (0, 0)),
        ],
        out_specs=[
            pl.BlockSpec((1, M, DOUT), lambda bi, idx: (bi, 0, 0)),
            pl.BlockSpec((1, M, 4), lambda bi, idx: (bi, 0, 0)),
        ],
        scratch_shapes=[pltpu.VMEM((M, DIN), jnp.float32)],
    )
    normed, auxo = pl.pallas_call(
        _gather_body,
        grid_spec=grid_spec,
        out_shape=[
            jax.ShapeDtypeStruct((B, M, DOUT), jnp.float32),
            jax.ShapeDtypeStruct((B, M, 4), jnp.float32),
        ],
    )(indices, features, aux, wt, b2, g2, be2)

    coords_out = auxo[..., :2]
    times_out = auxo[..., 2]
    pol_out = auxo[..., 3]
    return (normed, coords_out, times_out, pol_out)


# speedup vs baseline: 26.8194x; 1.5178x over previous
import jax
import jax.numpy as jnp
from jax.experimental import pallas as pl
from jax.experimental.pallas import tpu as pltpu

_RATIO = 0.25


def _fps_body(idx0_ref, l0_ref, px_ref, py_ref, pt_ref, out_ref,
              dist_ref, iota_ref):
    B, N = px_ref.shape
    LANE = 128
    NT = N // LANE
    M = out_ref.shape[1]
    lane_m = jax.lax.broadcasted_iota(jnp.int32, (B, M), 1)
    out_ref[...] = jnp.where(lane_m == 0, idx0_ref[...], 0)
    dist_ref[...] = jnp.full((B, N), jnp.inf, dtype=jnp.float32)
    iota_ref[...] = jax.lax.broadcasted_iota(
        jnp.int32, (B, N), 1).astype(jnp.float32)

    def chunks3(v):
        bits = jax.lax.bitcast_convert_type(v, jnp.int32)
        c0 = jax.lax.shift_right_logical(bits, 21)
        c1 = jax.lax.shift_right_logical(bits, 10) & 0x7FF
        c2 = bits & 0x3FF
        return [c0.astype(jnp.float32), c1.astype(jnp.float32),
                c2.astype(jnp.float32)]

    def recon(a, b, c):
        bits = ((a & 0x7FF) << 21) | ((b & 0x7FF) << 10) | (c & 0x3FF)
        return jax.lax.bitcast_convert_type(bits, jnp.float32)

    def body(i, carry):
        lx, ly, lt = carry
        pxv = px_ref[...]
        pyv = py_ref[...]
        ptv = pt_ref[...]
        dx = pxv - lx
        s = dx * dx
        dy = pyv - ly
        s = s + dy * dy
        dt = ptv - lt
        s = s + dt * dt
        d = jnp.sqrt(s + 1e-08)
        dmin = jnp.minimum(dist_ref[...], d)
        dist_ref[...] = dmin
        m = jnp.max(dmin, axis=1, keepdims=True)
        key = jnp.where(dmin == m, iota_ref[...], float(N))
        tiles = [(key[:, j * LANE:(j + 1) * LANE],
                  pxv[:, j * LANE:(j + 1) * LANE],
                  pyv[:, j * LANE:(j + 1) * LANE],
                  ptv[:, j * LANE:(j + 1) * LANE]) for j in range(NT)]
        while len(tiles) > 1:
            nxt = []
            for a, b in zip(tiles[0::2], tiles[1::2]):
                c = a[0] <= b[0]
                nxt.append(tuple(jnp.where(c, u, v)
                                 for u, v in zip(a, b)))
            tiles = nxt
        fkey, fpx, fpy, fpt = tiles[0]
        kb = fkey * 2048.0
        packs = [kb + c for v in (fpx, fpy, fpt) for c in chunks3(v)]
        r = [jnp.min(p, axis=1, keepdims=True).astype(jnp.int32)
             for p in packs]
        ki = jax.lax.shift_right_logical(r[0], 11)
        nlx = recon(r[0], r[1], r[2])
        nly = recon(r[3], r[4], r[5])
        nlt = recon(r[6], r[7], r[8])
        out_ref[...] = jnp.where(lane_m == i, ki, out_ref[...])
        return (nlx, nly, nlt)

    l0 = l0_ref[...]
    jax.lax.fori_loop(1, M, body,
                      (l0[:, 0:1], l0[:, 1:2], l0[:, 2:3]), unroll=4)


def _gather_body(idx_ref, feat_ref, aux_ref, wt_ref, b_ref, g_ref, be_ref,
                 out_ref, auxo_ref, gbuf):
    bi = pl.program_id(0)
    M = out_ref.shape[1]

    def copy_row(r, carry):
        k = idx_ref[bi, r]
        gbuf[pl.ds(r, 1), :] = feat_ref[0, pl.ds(k, 1), :]
        auxo_ref[0, pl.ds(r, 1), :] = aux_ref[0, pl.ds(k, 1), :]
        return carry

    jax.lax.fori_loop(0, M, copy_row, 0, unroll=8)
    feats = gbuf[...]
    proj = jnp.dot(feats, wt_ref[...], preferred_element_type=jnp.float32,
                   precision=jax.lax.Precision.HIGHEST) + b_ref[...]
    mean = jnp.mean(proj, axis=1, keepdims=True)
    var = jnp.mean((proj - mean) ** 2, axis=1, keepdims=True)
    normed = (proj - mean) / jnp.sqrt(var + 1e-05) * g_ref[...] + be_ref[...]
    out_ref[...] = normed[None]


def kernel(features, coords, times, polarities, W, b, gamma, beta):
    B, N, DIN = features.shape
    DOUT = W.shape[0]
    M = max(int(N * _RATIO), 1)

    px = coords[..., 0]
    py = coords[..., 1]
    pt = times
    idx0 = jax.random.randint(jax.random.key(1), (B,), 0, N).astype(jnp.int32)
    l0 = jnp.concatenate([
        jnp.take_along_axis(px, idx0[:, None], axis=1),
        jnp.take_along_axis(py, idx0[:, None], axis=1),
        jnp.take_along_axis(pt, idx0[:, None], axis=1),
        jnp.zeros((B, 1), jnp.float32),
    ], axis=1)
    idx0 = idx0[:, None]

    indices = pl.pallas_call(
        _fps_body,
        out_shape=jax.ShapeDtypeStruct((B, M), jnp.int32),
        scratch_shapes=[pltpu.VMEM((B, N), jnp.float32),
                        pltpu.VMEM((B, N), jnp.float32)],
    )(idx0, l0, px, py, pt)

    aux = jnp.stack([px, py, pt, polarities], axis=-1)
    wt = W.T
    b2 = b[None, :]
    g2 = gamma[None, :]
    be2 = beta[None, :]

    grid_spec = pltpu.PrefetchScalarGridSpec(
        num_scalar_prefetch=1,
        grid=(B,),
        in_specs=[
            pl.BlockSpec((1, N, DIN), lambda bi, idx: (bi, 0, 0)),
            pl.BlockSpec((1, N, 4), lambda bi, idx: (bi, 0, 0)),
            pl.BlockSpec((DIN, DOUT), lambda bi, idx: (0, 0)),
            pl.BlockSpec((1, DOUT), lambda bi, idx: (0, 0)),
            pl.BlockSpec((1, DOUT), lambda bi, idx: (0, 0)),
            pl.BlockSpec((1, DOUT), lambda bi, idx: (0, 0)),
        ],
        out_specs=[
            pl.BlockSpec((1, M, DOUT), lambda bi, idx: (bi, 0, 0)),
            pl.BlockSpec((1, M, 4), lambda bi, idx: (bi, 0, 0)),
        ],
        scratch_shapes=[pltpu.VMEM((M, DIN), jnp.float32)],
    )
    normed, auxo = pl.pallas_call(
        _gather_body,
        grid_spec=grid_spec,
        out_shape=[
            jax.ShapeDtypeStruct((B, M, DOUT), jnp.float32),
            jax.ShapeDtypeStruct((B, M, 4), jnp.float32),
        ],
    )(indices, features, aux, wt, b2, g2, be2)

    coords_out = auxo[..., :2]
    times_out = auxo[..., 2]
    pol_out = auxo[..., 3]
    return (normed, coords_out, times_out, pol_out)


# speedup vs baseline: 29.9194x; 1.1156x over previous
import jax
import jax.numpy as jnp
from jax.experimental import pallas as pl
from jax.experimental.pallas import tpu as pltpu
from jax.experimental.pallas import tpu_sc as plsc

_RATIO = 0.25
_GATHER_WINDOW = 128


def _sc_gather(flat_feat, flat_idx):
    n_idx = flat_idx.shape[1]
    din = flat_feat.shape[1]
    mesh = plsc.VectorSubcoreMesh(core_axis_name="core",
                                  subcore_axis_name="subcore")

    @pl.kernel(out_type=jax.ShapeDtypeStruct((n_idx, din), flat_feat.dtype),
               mesh=mesh)
    def gather_kernel(feat_hbm, i_hbm, fo_hbm):
        def body(i_vmem, fo_vmem):
            pltpu.sync_copy(feat_hbm.at[i_vmem.at[0]], fo_vmem)

        pltpu.emit_pipeline(
            body,
            grid=(n_idx // _GATHER_WINDOW,),
            in_specs=[pl.BlockSpec((1, _GATHER_WINDOW),
                                   index_map=lambda i: (0, i))],
            out_specs=[pl.BlockSpec((_GATHER_WINDOW, din),
                                    index_map=lambda i: (i, 0))],
            core_axis_name="subcore",
            dimension_semantics=(pltpu.PARALLEL,),
        )(i_hbm, fo_hbm)

    return gather_kernel(flat_feat, flat_idx)


def _fps_body(idx0_ref, l0_ref, px_ref, py_ref, pt_ref, pp_ref, out_ref,
              xo_ref, yo_ref, to_ref, po_ref, dist_ref, iota_ref):
    B, N = px_ref.shape
    LANE = 128
    NT = N // LANE
    M = out_ref.shape[1]
    lane_m = jax.lax.broadcasted_iota(jnp.int32, (B, M), 1)
    l0 = l0_ref[...]
    out_ref[...] = jnp.where(lane_m == 0, idx0_ref[...], 0)
    xo_ref[...] = jnp.where(lane_m == 0, l0[:, 0:1], 0.0)
    yo_ref[...] = jnp.where(lane_m == 0, l0[:, 1:2], 0.0)
    to_ref[...] = jnp.where(lane_m == 0, l0[:, 2:3], 0.0)
    po_ref[...] = jnp.where(lane_m == 0, l0[:, 3:4], 0.0)
    dist_ref[...] = jnp.full((B, N), jnp.inf, dtype=jnp.float32)
    iota_ref[...] = jax.lax.broadcasted_iota(
        jnp.int32, (B, N), 1).astype(jnp.float32)

    def chunks3(v):
        bits = jax.lax.bitcast_convert_type(v, jnp.int32)
        c0 = jax.lax.shift_right_logical(bits, 21)
        c1 = jax.lax.shift_right_logical(bits, 10) & 0x7FF
        c2 = bits & 0x3FF
        return [c0.astype(jnp.float32), c1.astype(jnp.float32),
                c2.astype(jnp.float32)]

    def recon(a, b, c):
        bits = ((a & 0x7FF) << 21) | ((b & 0x7FF) << 10) | (c & 0x3FF)
        return jax.lax.bitcast_convert_type(bits, jnp.float32)

    def body(i, carry):
        lx, ly, lt = carry
        pxv = px_ref[...]
        pyv = py_ref[...]
        ptv = pt_ref[...]
        dx = pxv - lx
        s = dx * dx
        dy = pyv - ly
        s = s + dy * dy
        dt = ptv - lt
        s = s + dt * dt
        d = jnp.sqrt(s + 1e-08)
        dmin = jnp.minimum(dist_ref[...], d)
        dist_ref[...] = dmin
        m = jnp.max(dmin, axis=1, keepdims=True)
        key = jnp.where(dmin == m, iota_ref[...], float(N))
        ppv = pp_ref[...]
        tiles = [(key[:, j * LANE:(j + 1) * LANE],
                  pxv[:, j * LANE:(j + 1) * LANE],
                  pyv[:, j * LANE:(j + 1) * LANE],
                  ptv[:, j * LANE:(j + 1) * LANE],
                  ppv[:, j * LANE:(j + 1) * LANE]) for j in range(NT)]
        while len(tiles) > 1:
            nxt = []
            for a, b in zip(tiles[0::2], tiles[1::2]):
                c = a[0] <= b[0]
                nxt.append(tuple(jnp.where(c, u, v)
                                 for u, v in zip(a, b)))
            tiles = nxt
        fkey, fpx, fpy, fpt, fpp = tiles[0]
        kb = fkey * 2048.0
        packs = [kb + c for v in (fpx, fpy, fpt, fpp) for c in chunks3(v)]
        r = [jnp.min(p, axis=1, keepdims=True).astype(jnp.int32)
             for p in packs]
        ki = jax.lax.shift_right_logical(r[0], 11)
        nlx = recon(r[0], r[1], r[2])
        nly = recon(r[3], r[4], r[5])
        nlt = recon(r[6], r[7], r[8])
        npp = recon(r[9], r[10], r[11])
        sel_i = lane_m == i
        out_ref[...] = jnp.where(sel_i, ki, out_ref[...])
        xo_ref[...] = jnp.where(sel_i, nlx, xo_ref[...])
        yo_ref[...] = jnp.where(sel_i, nly, yo_ref[...])
        to_ref[...] = jnp.where(sel_i, nlt, to_ref[...])
        po_ref[...] = jnp.where(sel_i, npp, po_ref[...])
        return (nlx, nly, nlt)

    jax.lax.fori_loop(1, M, body,
                      (l0[:, 0:1], l0[:, 1:2], l0[:, 2:3]), unroll=4)


def _proj_body(feat_ref, wt_ref, b_ref, g_ref, be_ref, out_ref):
    feats = feat_ref[0]
    proj = jnp.dot(feats, wt_ref[...], preferred_element_type=jnp.float32,
                   precision=jax.lax.Precision.HIGHEST) + b_ref[...]
    mean = jnp.mean(proj, axis=1, keepdims=True)
    var = jnp.mean((proj - mean) ** 2, axis=1, keepdims=True)
    normed = (proj - mean) / jnp.sqrt(var + 1e-05) * g_ref[...] + be_ref[...]
    out_ref[...] = normed[None]


def kernel(features, coords, times, polarities, W, b, gamma, beta):
    B, N, DIN = features.shape
    DOUT = W.shape[0]
    M = max(int(N * _RATIO), 1)

    px = coords[..., 0]
    py = coords[..., 1]
    pt = times
    idx0 = jax.random.randint(jax.random.key(1), (B,), 0, N).astype(jnp.int32)
    l0 = jnp.concatenate([
        jnp.take_along_axis(px, idx0[:, None], axis=1),
        jnp.take_along_axis(py, idx0[:, None], axis=1),
        jnp.take_along_axis(pt, idx0[:, None], axis=1),
        jnp.take_along_axis(polarities, idx0[:, None], axis=1),
    ], axis=1)
    idx0 = idx0[:, None]

    indices, xo, yo, to, po = pl.pallas_call(
        _fps_body,
        out_shape=[
            jax.ShapeDtypeStruct((B, M), jnp.int32),
            jax.ShapeDtypeStruct((B, M), jnp.float32),
            jax.ShapeDtypeStruct((B, M), jnp.float32),
            jax.ShapeDtypeStruct((B, M), jnp.float32),
            jax.ShapeDtypeStruct((B, M), jnp.float32),
        ],
        scratch_shapes=[pltpu.VMEM((B, N), jnp.float32),
                        pltpu.VMEM((B, N), jnp.float32)],
    )(idx0, l0, px, py, pt, polarities)

    flat_feat = features.reshape(B * N, DIN)
    flat_idx = (indices + jnp.arange(B, dtype=jnp.int32)[:, None] * N
                ).reshape(1, B * M)
    feat_g = _sc_gather(flat_feat, flat_idx).reshape(B, M, DIN)

    wt = W.T
    b2 = b[None, :]
    g2 = gamma[None, :]
    be2 = beta[None, :]

    normed = pl.pallas_call(
        _proj_body,
        grid=(B,),
        in_specs=[
            pl.BlockSpec((1, M, DIN), lambda bi: (bi, 0, 0)),
            pl.BlockSpec((DIN, DOUT), lambda bi: (0, 0)),
            pl.BlockSpec((1, DOUT), lambda bi: (0, 0)),
            pl.BlockSpec((1, DOUT), lambda bi: (0, 0)),
            pl.BlockSpec((1, DOUT), lambda bi: (0, 0)),
        ],
        out_specs=pl.BlockSpec((1, M, DOUT), lambda bi: (bi, 0, 0)),
        out_shape=jax.ShapeDtypeStruct((B, M, DOUT), jnp.float32),
    )(feat_g, wt, b2, g2, be2)

    coords_out = jnp.stack([xo, yo], axis=-1)
    return (normed, coords_out, to, po)


# speedup vs baseline: 33.8700x; 1.1320x over previous
import jax
import jax.numpy as jnp
from jax.experimental import pallas as pl
from jax.experimental.pallas import tpu as pltpu
from jax.experimental.pallas import tpu_sc as plsc

_RATIO = 0.25
_GATHER_WINDOW = 128


def _sc_gather(flat_feat, flat_idx):
    n_idx = flat_idx.shape[1]
    din = flat_feat.shape[1]
    mesh = plsc.VectorSubcoreMesh(core_axis_name="core",
                                  subcore_axis_name="subcore")

    @pl.kernel(out_type=jax.ShapeDtypeStruct((n_idx, din), flat_feat.dtype),
               mesh=mesh)
    def gather_kernel(feat_hbm, i_hbm, fo_hbm):
        def body(i_vmem, fo_vmem):
            pltpu.sync_copy(feat_hbm.at[i_vmem.at[0]], fo_vmem)

        pltpu.emit_pipeline(
            body,
            grid=(n_idx // _GATHER_WINDOW,),
            in_specs=[pl.BlockSpec((1, _GATHER_WINDOW),
                                   index_map=lambda i: (0, i))],
            out_specs=[pl.BlockSpec((_GATHER_WINDOW, din),
                                    index_map=lambda i: (i, 0))],
            core_axis_name="subcore",
            dimension_semantics=(pltpu.PARALLEL,),
        )(i_hbm, fo_hbm)

    return gather_kernel(flat_feat, flat_idx)


def _fps_body(idx0_ref, l0_ref, px_ref, py_ref, pt_ref, pp_ref, out_ref,
              xo_ref, yo_ref, to_ref, po_ref, dist_ref, iota_ref):
    B, N = px_ref.shape
    LANE = 128
    NT = N // LANE
    M = out_ref.shape[1]
    lane_m = jax.lax.broadcasted_iota(jnp.int32, (B, M), 1)
    l0 = l0_ref[...]
    out_ref[...] = jnp.where(lane_m == 0, idx0_ref[...], 0)
    xo_ref[...] = jnp.where(lane_m == 0, l0[:, 0:1], 0.0)
    yo_ref[...] = jnp.where(lane_m == 0, l0[:, 1:2], 0.0)
    to_ref[...] = jnp.where(lane_m == 0, l0[:, 2:3], 0.0)
    po_ref[...] = jnp.where(lane_m == 0, l0[:, 3:4], 0.0)
    dist_ref[...] = jnp.full((B, N), jnp.inf, dtype=jnp.float32)
    iota_ref[...] = jax.lax.broadcasted_iota(
        jnp.int32, (B, N), 1).astype(jnp.float32)

    def chunks3(v):
        bits = jax.lax.bitcast_convert_type(v, jnp.int32)
        c0 = jax.lax.shift_right_logical(bits, 21)
        c1 = jax.lax.shift_right_logical(bits, 10) & 0x7FF
        c2 = bits & 0x3FF
        return [c0.astype(jnp.float32), c1.astype(jnp.float32),
                c2.astype(jnp.float32)]

    def recon(a, b, c):
        bits = ((a & 0x7FF) << 21) | ((b & 0x7FF) << 10) | (c & 0x3FF)
        return jax.lax.bitcast_convert_type(bits, jnp.float32)

    def body(i, carry):
        lx, ly, lt = carry
        pxv = px_ref[...]
        pyv = py_ref[...]
        ptv = pt_ref[...]
        dx = pxv - lx
        s = dx * dx
        dy = pyv - ly
        s = s + dy * dy
        dt = ptv - lt
        s = s + dt * dt
        d = jnp.sqrt(s + 1e-08)
        dmin = jnp.minimum(dist_ref[...], d)
        dist_ref[...] = dmin
        iov = iota_ref[...]
        ppv = pp_ref[...]
        tiles = [(dmin[:, j * LANE:(j + 1) * LANE],
                  iov[:, j * LANE:(j + 1) * LANE],
                  pxv[:, j * LANE:(j + 1) * LANE],
                  pyv[:, j * LANE:(j + 1) * LANE],
                  ptv[:, j * LANE:(j + 1) * LANE],
                  ppv[:, j * LANE:(j + 1) * LANE]) for j in range(NT)]
        while len(tiles) > 1:
            nxt = []
            for a, b in zip(tiles[0::2], tiles[1::2]):
                c = a[0] >= b[0]
                nxt.append(tuple(jnp.where(c, u, v)
                                 for u, v in zip(a, b)))
            tiles = nxt
        fd, fkey, fpx, fpy, fpt, fpp = tiles[0]
        m = jnp.max(fd, axis=1, keepdims=True)
        fkey = jnp.where(fd == m, fkey, float(N))
        kb = fkey * 2048.0
        packs = [kb + c for v in (fpx, fpy, fpt, fpp) for c in chunks3(v)]
        r = [jnp.min(p, axis=1, keepdims=True).astype(jnp.int32)
             for p in packs]
        ki = jax.lax.shift_right_logical(r[0], 11)
        nlx = recon(r[0], r[1], r[2])
        nly = recon(r[3], r[4], r[5])
        nlt = recon(r[6], r[7], r[8])
        npp = recon(r[9], r[10], r[11])
        sel_i = lane_m == i
        out_ref[...] = jnp.where(sel_i, ki, out_ref[...])
        xo_ref[...] = jnp.where(sel_i, nlx, xo_ref[...])
        yo_ref[...] = jnp.where(sel_i, nly, yo_ref[...])
        to_ref[...] = jnp.where(sel_i, nlt, to_ref[...])
        po_ref[...] = jnp.where(sel_i, npp, po_ref[...])
        return (nlx, nly, nlt)

    jax.lax.fori_loop(1, M, body,
                      (l0[:, 0:1], l0[:, 1:2], l0[:, 2:3]), unroll=4)


def _proj_body(feat_ref, wt_ref, b_ref, g_ref, be_ref, out_ref):
    feats = feat_ref[0]
    proj = jnp.dot(feats, wt_ref[...], preferred_element_type=jnp.float32,
                   precision=jax.lax.Precision.HIGHEST) + b_ref[...]
    mean = jnp.mean(proj, axis=1, keepdims=True)
    var = jnp.mean((proj - mean) ** 2, axis=1, keepdims=True)
    normed = (proj - mean) / jnp.sqrt(var + 1e-05) * g_ref[...] + be_ref[...]
    out_ref[...] = normed[None]


def kernel(features, coords, times, polarities, W, b, gamma, beta):
    B, N, DIN = features.shape
    DOUT = W.shape[0]
    M = max(int(N * _RATIO), 1)

    px = coords[..., 0]
    py = coords[..., 1]
    pt = times
    idx0 = jax.random.randint(jax.random.key(1), (B,), 0, N).astype(jnp.int32)
    l0 = jnp.concatenate([
        jnp.take_along_axis(px, idx0[:, None], axis=1),
        jnp.take_along_axis(py, idx0[:, None], axis=1),
        jnp.take_along_axis(pt, idx0[:, None], axis=1),
        jnp.take_along_axis(polarities, idx0[:, None], axis=1),
    ], axis=1)
    idx0 = idx0[:, None]

    indices, xo, yo, to, po = pl.pallas_call(
        _fps_body,
        out_shape=[
            jax.ShapeDtypeStruct((B, M), jnp.int32),
            jax.ShapeDtypeStruct((B, M), jnp.float32),
            jax.ShapeDtypeStruct((B, M), jnp.float32),
            jax.ShapeDtypeStruct((B, M), jnp.float32),
            jax.ShapeDtypeStruct((B, M), jnp.float32),
        ],
        scratch_shapes=[pltpu.VMEM((B, N), jnp.float32),
                        pltpu.VMEM((B, N), jnp.float32)],
    )(idx0, l0, px, py, pt, polarities)

    flat_feat = features.reshape(B * N, DIN)
    flat_idx = (indices + jnp.arange(B, dtype=jnp.int32)[:, None] * N
                ).reshape(1, B * M)
    feat_g = _sc_gather(flat_feat, flat_idx).reshape(B, M, DIN)

    wt = W.T
    b2 = b[None, :]
    g2 = gamma[None, :]
    be2 = beta[None, :]

    normed = pl.pallas_call(
        _proj_body,
        grid=(B,),
        in_specs=[
            pl.BlockSpec((1, M, DIN), lambda bi: (bi, 0, 0)),
            pl.BlockSpec((DIN, DOUT), lambda bi: (0, 0)),
            pl.BlockSpec((1, DOUT), lambda bi: (0, 0)),
            pl.BlockSpec((1, DOUT), lambda bi: (0, 0)),
            pl.BlockSpec((1, DOUT), lambda bi: (0, 0)),
        ],
        out_specs=pl.BlockSpec((1, M, DOUT), lambda bi: (bi, 0, 0)),
        out_shape=jax.ShapeDtypeStruct((B, M, DOUT), jnp.float32),
    )(feat_g, wt, b2, g2, be2)

    coords_out = jnp.stack([xo, yo], axis=-1)
    return (normed, coords_out, to, po)


# speedup vs baseline: 34.8613x; 1.0293x over previous
import jax
import jax.numpy as jnp
from jax.experimental import pallas as pl
from jax.experimental.pallas import tpu as pltpu
from jax.experimental.pallas import tpu_sc as plsc

_RATIO = 0.25
_GATHER_WINDOW = 128


def _sc_gather(flat_feat, flat_idx):
    n_idx = flat_idx.shape[1]
    din = flat_feat.shape[1]
    mesh = plsc.VectorSubcoreMesh(core_axis_name="core",
                                  subcore_axis_name="subcore")

    @pl.kernel(out_type=jax.ShapeDtypeStruct((n_idx, din), flat_feat.dtype),
               mesh=mesh)
    def gather_kernel(feat_hbm, i_hbm, fo_hbm):
        def body(i_vmem, fo_vmem):
            pltpu.sync_copy(feat_hbm.at[i_vmem.at[0]], fo_vmem)

        pltpu.emit_pipeline(
            body,
            grid=(n_idx // _GATHER_WINDOW,),
            in_specs=[pl.BlockSpec((1, _GATHER_WINDOW),
                                   index_map=lambda i: (0, i))],
            out_specs=[pl.BlockSpec((_GATHER_WINDOW, din),
                                    index_map=lambda i: (i, 0))],
            core_axis_name="subcore",
            dimension_semantics=(pltpu.PARALLEL,),
        )(i_hbm, fo_hbm)

    return gather_kernel(flat_feat, flat_idx)


def _fps_body(idx0_ref, l0_ref, px_ref, py_ref, pt_ref, pp_ref, out_ref,
              xo_ref, yo_ref, to_ref, po_ref, dist_ref, iota_ref):
    B, N = px_ref.shape
    LANE = 128
    NT = N // LANE
    M = out_ref.shape[1]
    lane_m = jax.lax.broadcasted_iota(jnp.int32, (B, M), 1)
    l0 = l0_ref[...]
    out_ref[...] = jnp.where(lane_m == 0, idx0_ref[...], 0)
    xo_ref[...] = jnp.where(lane_m == 0, l0[:, 0:1], 0.0)
    yo_ref[...] = jnp.where(lane_m == 0, l0[:, 1:2], 0.0)
    to_ref[...] = jnp.where(lane_m == 0, l0[:, 2:3], 0.0)
    po_ref[...] = jnp.where(lane_m == 0, l0[:, 3:4], 0.0)
    dist_ref[...] = jnp.full((B, N), jnp.inf, dtype=jnp.float32)
    iota_ref[...] = jax.lax.broadcasted_iota(
        jnp.int32, (B, N), 1).astype(jnp.float32)

    def chunks3(v):
        bits = jax.lax.bitcast_convert_type(v, jnp.int32)
        c0 = jax.lax.shift_right_logical(bits, 21)
        c1 = jax.lax.shift_right_logical(bits, 10) & 0x7FF
        c2 = bits & 0x3FF
        return [c0.astype(jnp.float32), c1.astype(jnp.float32),
                c2.astype(jnp.float32)]

    def recon(a, b, c):
        bits = ((a & 0x7FF) << 21) | ((b & 0x7FF) << 10) | (c & 0x3FF)
        return jax.lax.bitcast_convert_type(bits, jnp.float32)

    def body(i, carry):
        lx, ly, lt = carry
        pxv = px_ref[...]
        pyv = py_ref[...]
        ptv = pt_ref[...]
        dx = pxv - lx
        s = dx * dx
        dy = pyv - ly
        s = s + dy * dy
        dt = ptv - lt
        s = s + dt * dt
        d = jnp.sqrt(s + 1e-08)
        dmin = jnp.minimum(dist_ref[...], d)
        dist_ref[...] = dmin
        iov = iota_ref[...]
        ppv = pp_ref[...]
        tiles = [(dmin[:, j * LANE:(j + 1) * LANE],
                  iov[:, j * LANE:(j + 1) * LANE],
                  pxv[:, j * LANE:(j + 1) * LANE],
                  pyv[:, j * LANE:(j + 1) * LANE],
                  ptv[:, j * LANE:(j + 1) * LANE],
                  ppv[:, j * LANE:(j + 1) * LANE]) for j in range(NT)]
        while len(tiles) > 1:
            nxt = []
            for a, b in zip(tiles[0::2], tiles[1::2]):
                c = a[0] >= b[0]
                nxt.append(tuple(jnp.where(c, u, v)
                                 for u, v in zip(a, b)))
            tiles = nxt
        fd, fkey, fpx, fpy, fpt, fpp = tiles[0]
        m = jnp.max(fd, axis=1, keepdims=True)
        fkey = jnp.where(fd == m, fkey, float(N))
        kb = fkey * 2048.0
        packs = [kb + c for v in (fpx, fpy, fpt, fpp) for c in chunks3(v)]
        r = [jnp.min(p, axis=1, keepdims=True).astype(jnp.int32)
             for p in packs]
        ki = jax.lax.shift_right_logical(r[0], 11)
        nlx = recon(r[0], r[1], r[2])
        nly = recon(r[3], r[4], r[5])
        nlt = recon(r[6], r[7], r[8])
        npp = recon(r[9], r[10], r[11])
        sel_i = lane_m == i
        out_ref[...] = jnp.where(sel_i, ki, out_ref[...])
        xo_ref[...] = jnp.where(sel_i, nlx, xo_ref[...])
        yo_ref[...] = jnp.where(sel_i, nly, yo_ref[...])
        to_ref[...] = jnp.where(sel_i, nlt, to_ref[...])
        po_ref[...] = jnp.where(sel_i, npp, po_ref[...])
        return (nlx, nly, nlt)

    jax.lax.fori_loop(1, M, body,
                      (l0[:, 0:1], l0[:, 1:2], l0[:, 2:3]), unroll=8)


def _proj_body(feat_ref, wt_ref, b_ref, g_ref, be_ref, out_ref):
    feats = feat_ref[0]
    proj = jnp.dot(feats, wt_ref[...], preferred_element_type=jnp.float32,
                   precision=jax.lax.Precision.HIGHEST) + b_ref[...]
    mean = jnp.mean(proj, axis=1, keepdims=True)
    var = jnp.mean((proj - mean) ** 2, axis=1, keepdims=True)
    normed = (proj - mean) / jnp.sqrt(var + 1e-05) * g_ref[...] + be_ref[...]
    out_ref[...] = normed[None]


def kernel(features, coords, times, polarities, W, b, gamma, beta):
    B, N, DIN = features.shape
    DOUT = W.shape[0]
    M = max(int(N * _RATIO), 1)

    px = coords[..., 0]
    py = coords[..., 1]
    pt = times
    idx0 = jax.random.randint(jax.random.key(1), (B,), 0, N).astype(jnp.int32)
    l0 = jnp.concatenate([
        jnp.take_along_axis(px, idx0[:, None], axis=1),
        jnp.take_along_axis(py, idx0[:, None], axis=1),
        jnp.take_along_axis(pt, idx0[:, None], axis=1),
        jnp.take_along_axis(polarities, idx0[:, None], axis=1),
    ], axis=1)
    idx0 = idx0[:, None]

    indices, xo, yo, to, po = pl.pallas_call(
        _fps_body,
        out_shape=[
            jax.ShapeDtypeStruct((B, M), jnp.int32),
            jax.ShapeDtypeStruct((B, M), jnp.float32),
            jax.ShapeDtypeStruct((B, M), jnp.float32),
            jax.ShapeDtypeStruct((B, M), jnp.float32),
            jax.ShapeDtypeStruct((B, M), jnp.float32),
        ],
        scratch_shapes=[pltpu.VMEM((B, N), jnp.float32),
                        pltpu.VMEM((B, N), jnp.float32)],
    )(idx0, l0, px, py, pt, polarities)

    flat_feat = features.reshape(B * N, DIN)
    flat_idx = (indices + jnp.arange(B, dtype=jnp.int32)[:, None] * N
                ).reshape(1, B * M)
    feat_g = _sc_gather(flat_feat, flat_idx).reshape(B, M, DIN)

    wt = W.T
    b2 = b[None, :]
    g2 = gamma[None, :]
    be2 = beta[None, :]

    normed = pl.pallas_call(
        _proj_body,
        grid=(B,),
        in_specs=[
            pl.BlockSpec((1, M, DIN), lambda bi: (bi, 0, 0)),
            pl.BlockSpec((DIN, DOUT), lambda bi: (0, 0)),
            pl.BlockSpec((1, DOUT), lambda bi: (0, 0)),
            pl.BlockSpec((1, DOUT), lambda bi: (0, 0)),
            pl.BlockSpec((1, DOUT), lambda bi: (0, 0)),
        ],
        out_specs=pl.BlockSpec((1, M, DOUT), lambda bi: (bi, 0, 0)),
        out_shape=jax.ShapeDtypeStruct((B, M, DOUT), jnp.float32),
    )(feat_g, wt, b2, g2, be2)

    coords_out = jnp.stack([xo, yo], axis=-1)
    return (normed, coords_out, to, po)
